# Initial kernel scaffold; baseline (speedup 1.0000x reference)
import jax
import jax.numpy as jnp
from jax.experimental import pallas as pl


def kernel(feat, edge_index, etypes, W, coeff, h_bias, loop_weight):
    raise NotImplementedError("write your pallas kernel here")


# speedup vs baseline: 9.9466x; 9.9466x over previous
import functools

import jax
import jax.numpy as jnp
from jax import lax
from jax.experimental import pallas as pl
from jax.experimental.pallas import tpu as pltpu
from jax.experimental.pallas import tpu_sc as plsc

N = 10000
E = 320000
F = 128
FH = F // 2
R = 16
NB = 4

NC = 2
NS = 16
EPS = E // NS
K = 800
NBATCH = EPS // K
NPAD = 10112
ROWS_PER_TILE = NPAD // NS

TN = 2000
NT = N // TN


def _table_body(coeff_ref, feat_ref, w_ref, out_ref):
    r = pl.program_id(0)
    h = pl.program_id(2)
    wr = coeff_ref[r, 0] * w_ref[0]
    for b in range(1, NB):
        wr = wr + coeff_ref[r, b] * w_ref[b]
    d = jnp.dot(feat_ref[...], wr, preferred_element_type=jnp.float32)

    @pl.when(h == 0)
    def _():
        out_ref[...] = d[:, :FH]

    @pl.when(h == 1)
    def _():
        out_ref[...] = d[:, FH:]


def _build_table(feat, W, coeff):
    return pl.pallas_call(
        _table_body,
        grid=(R, NT, 2),
        in_specs=[
            pl.BlockSpec(memory_space=pltpu.SMEM),
            pl.BlockSpec((TN, F), lambda r, n, h: (n, 0)),
            pl.BlockSpec((NB, F, F), lambda r, n, h: (0, 0, 0)),
        ],
        out_specs=pl.BlockSpec(
            (TN, FH), lambda r, n, h: (h * R * NT + r * NT + n, 0)),
        out_shape=jax.ShapeDtypeStruct((2 * R * N, FH), jnp.float32),
    )(coeff, feat, W)


_MESH = plsc.VectorSubcoreMesh(core_axis_name="c", subcore_axis_name="s")


@functools.partial(
    pl.kernel,
    mesh=_MESH,
    compiler_params=pltpu.CompilerParams(use_tc_tiling_on_sc=False),
    out_type=jax.ShapeDtypeStruct((NC, NPAD, FH), jnp.float32),
    scratch_types=[
        pltpu.VMEM((K,), jnp.int32),
        pltpu.VMEM((K,), jnp.int32),
        pltpu.VMEM((K,), jnp.int32),
        pltpu.VMEM((K,), jnp.int32),
        pltpu.VMEM((K, FH), jnp.float32),
        pltpu.VMEM_SHARED((NPAD, FH), jnp.float32),
        pltpu.SemaphoreType.DMA,
    ],
)
def _edge_scatter(t_hbm, src_hbm, et_hbm, dst_hbm, zeros_hbm, out_hbm,
                  srcv, etv, dstv, idxv, rows, acc, sem):
    c = lax.axis_index("c")
    s = lax.axis_index("s")

    pltpu.sync_copy(zeros_hbm.at[pl.ds(s * ROWS_PER_TILE, ROWS_PER_TILE)],
                    acc.at[pl.ds(s * ROWS_PER_TILE, ROWS_PER_TILE)])
    plsc.subcore_barrier()

    base = s * EPS
    cbase = c * (R * N)

    def batch(i, carry):
        off = base + i * K
        pltpu.sync_copy(src_hbm.at[pl.ds(off, K)], srcv)
        pltpu.sync_copy(et_hbm.at[pl.ds(off, K)], etv)
        pltpu.sync_copy(dst_hbm.at[pl.ds(off, K)], dstv)
        for j in range(K // 16):
            sl = pl.ds(j * 16, 16)
            idxv[sl] = etv[sl] * N + srcv[sl] + cbase
        pltpu.async_copy(t_hbm.at[idxv], rows, sem).wait()
        pltpu.sync_copy(rows, acc.at[dstv], add=True)
        return carry

    lax.fori_loop(0, NBATCH, batch, 0)

    plsc.subcore_barrier()
    pltpu.sync_copy(acc.at[pl.ds(s * ROWS_PER_TILE, ROWS_PER_TILE)],
                    out_hbm.at[c, pl.ds(s * ROWS_PER_TILE, ROWS_PER_TILE)])


def _final_body(p_ref, feat_ref, lw_ref, bias_ref, out_ref):
    h = jnp.dot(feat_ref[...], lw_ref[...], preferred_element_type=jnp.float32)
    agg = jnp.concatenate([p_ref[0], p_ref[1]], axis=1)
    out_ref[...] = h + agg + bias_ref[...]


def _finalize(partials, feat, loop_weight, h_bias):
    return pl.pallas_call(
        _final_body,
        grid=(NT,),
        in_specs=[
            pl.BlockSpec((NC, TN, FH), lambda n: (0, n, 0)),
            pl.BlockSpec((TN, F), lambda n: (n, 0)),
            pl.BlockSpec((F, F), lambda n: (0, 0)),
            pl.BlockSpec((1, F), lambda n: (0, 0)),
        ],
        out_specs=pl.BlockSpec((TN, F), lambda n: (n, 0)),
        out_shape=jax.ShapeDtypeStruct((N, F), jnp.float32),
    )(partials, feat, loop_weight, h_bias.reshape(1, F))


def kernel(feat, edge_index, etypes, W, coeff, h_bias, loop_weight):
    src = edge_index[0]
    dst = edge_index[1]
    zeros = jnp.zeros((NPAD, FH), jnp.float32)
    table = _build_table(feat, W, coeff)
    partials = _edge_scatter(table, src, etypes, dst, zeros)
    return _finalize(partials, feat, loop_weight, h_bias)


# speedup vs baseline: 18.9248x; 1.9026x over previous
import functools

import jax
import jax.numpy as jnp
from jax import lax
from jax.experimental import pallas as pl
from jax.experimental.pallas import tpu as pltpu
from jax.experimental.pallas import tpu_sc as plsc

N = 10000
E = 320000
F = 128
FH = F // 2
R = 16
NB = 4

NC = 2
NS = 16
EPS = E // NS
K = 800
NBATCH = EPS // K
NPAD = 10112
ROWS_PER_TILE = NPAD // NS

TN = 2000
NT = N // TN


def _table_body(coeff_ref, feat_ref, w_ref, out_ref):
    r = pl.program_id(1)
    wr = coeff_ref[r, 0] * w_ref[0]
    for b in range(1, NB):
        wr = wr + coeff_ref[r, b] * w_ref[b]
    out_ref[...] = jnp.dot(feat_ref[...], wr, preferred_element_type=jnp.float32)


def _build_table(feat, W, coeff):
    return pl.pallas_call(
        _table_body,
        grid=(NT, R),
        in_specs=[
            pl.BlockSpec(memory_space=pltpu.SMEM),
            pl.BlockSpec((TN, F), lambda n, r: (n, 0)),
            pl.BlockSpec((NB, F, F), lambda n, r: (0, 0, 0)),
        ],
        out_specs=pl.BlockSpec((TN, F), lambda n, r: (r * NT + n, 0)),
        out_shape=jax.ShapeDtypeStruct((R * N, F), jnp.float32),
    )(coeff, feat, W)


_MESH = plsc.VectorSubcoreMesh(core_axis_name="c", subcore_axis_name="s")


@functools.partial(
    pl.kernel,
    mesh=_MESH,
    compiler_params=pltpu.CompilerParams(use_tc_tiling_on_sc=False),
    out_type=jax.ShapeDtypeStruct((NC, NPAD, FH), jnp.float32),
    scratch_types=[
        pltpu.VMEM((K,), jnp.int32),
        pltpu.VMEM((K,), jnp.int32),
        pltpu.VMEM((K,), jnp.int32),
        pltpu.VMEM((K,), jnp.int32),
        pltpu.VMEM((K, FH), jnp.float32),
        pltpu.VMEM_SHARED((NPAD, FH), jnp.float32),
        pltpu.SemaphoreType.DMA,
    ],
)
def _edge_scatter(t_hbm, ei_hbm, et_hbm, zeros_hbm, out_hbm,
                  srcv, etv, dstv, idxv, rows, acc, sem):
    c = lax.axis_index("c")
    s = lax.axis_index("s")

    pltpu.sync_copy(zeros_hbm.at[pl.ds(s * ROWS_PER_TILE, ROWS_PER_TILE)],
                    acc.at[pl.ds(s * ROWS_PER_TILE, ROWS_PER_TILE)])
    plsc.subcore_barrier()

    base = s * EPS

    def batch(i, carry):
        off = base + i * K
        pltpu.sync_copy(ei_hbm.at[0, pl.ds(off, K)], srcv)
        pltpu.sync_copy(et_hbm.at[pl.ds(off, K)], etv)
        pltpu.sync_copy(ei_hbm.at[1, pl.ds(off, K)], dstv)
        for j in range(K // 16):
            sl = pl.ds(j * 16, 16)
            idxv[sl] = (etv[sl] * N + srcv[sl]) * 2 + c
        pltpu.async_copy(t_hbm.at[idxv], rows, sem).wait()
        pltpu.sync_copy(rows, acc.at[dstv], add=True)
        return carry

    lax.fori_loop(0, NBATCH, batch, 0)

    plsc.subcore_barrier()
    pltpu.sync_copy(acc.at[pl.ds(s * ROWS_PER_TILE, ROWS_PER_TILE)],
                    out_hbm.at[c, pl.ds(s * ROWS_PER_TILE, ROWS_PER_TILE)])


def _final_body(p_ref, feat_ref, lw_ref, bias_ref, out_ref):
    h = jnp.dot(feat_ref[...], lw_ref[...], preferred_element_type=jnp.float32)
    agg = jnp.concatenate([p_ref[0], p_ref[1]], axis=1)
    out_ref[...] = h + agg + bias_ref[...]


def _finalize(partials, feat, loop_weight, h_bias):
    return pl.pallas_call(
        _final_body,
        grid=(NT,),
        in_specs=[
            pl.BlockSpec((NC, TN, FH), lambda n: (0, n, 0)),
            pl.BlockSpec((TN, F), lambda n: (n, 0)),
            pl.BlockSpec((F, F), lambda n: (0, 0)),
            pl.BlockSpec((1, F), lambda n: (0, 0)),
        ],
        out_specs=pl.BlockSpec((TN, F), lambda n: (n, 0)),
        out_shape=jax.ShapeDtypeStruct((N, F), jnp.float32),
    )(partials, feat, loop_weight, h_bias.reshape(1, F))


def kernel(feat, edge_index, etypes, W, coeff, h_bias, loop_weight):
    zeros = jnp.zeros((NPAD, FH), jnp.float32)
    table = _build_table(feat, W, coeff)
    table64 = table.reshape(2 * R * N, FH)
    partials = _edge_scatter(table64, edge_index, etypes, zeros)
    return _finalize(partials, feat, loop_weight, h_bias)


# speedup vs baseline: 20.8639x; 1.1025x over previous
import functools

import jax
import jax.numpy as jnp
from jax import lax
from jax.experimental import pallas as pl
from jax.experimental.pallas import tpu as pltpu
from jax.experimental.pallas import tpu_sc as plsc

N = 10000
E = 320000
F = 128
FH = F // 2
R = 16
NB = 4

NC = 2
NS = 16
EPS = E // NS
K = 400
NBATCH = EPS // K
NPAD = 10112
ROWS_PER_TILE = NPAD // NS

TN = 2000
NT = N // TN


def _table_body(coeff_ref, feat_ref, w_ref, out_ref):
    r = pl.program_id(1)
    wr = coeff_ref[r, 0] * w_ref[0]
    for b in range(1, NB):
        wr = wr + coeff_ref[r, b] * w_ref[b]
    out_ref[...] = jnp.dot(feat_ref[...], wr, preferred_element_type=jnp.float32)


def _build_table(feat, W, coeff):
    return pl.pallas_call(
        _table_body,
        grid=(NT, R),
        in_specs=[
            pl.BlockSpec(memory_space=pltpu.SMEM),
            pl.BlockSpec((TN, F), lambda n, r: (n, 0)),
            pl.BlockSpec((NB, F, F), lambda n, r: (0, 0, 0)),
        ],
        out_specs=pl.BlockSpec((TN, F), lambda n, r: (r * NT + n, 0)),
        out_shape=jax.ShapeDtypeStruct((R * N, F), jnp.float32),
    )(coeff, feat, W)


_MESH = plsc.VectorSubcoreMesh(core_axis_name="c", subcore_axis_name="s")


@functools.partial(
    pl.kernel,
    mesh=_MESH,
    compiler_params=pltpu.CompilerParams(use_tc_tiling_on_sc=False),
    out_type=jax.ShapeDtypeStruct((NC, NPAD, FH), jnp.float32),
    scratch_types=[
        pltpu.VMEM((K,), jnp.int32),
        pltpu.VMEM((K,), jnp.int32),
        pltpu.VMEM((K,), jnp.int32),
        pltpu.VMEM((K,), jnp.int32),
        pltpu.VMEM((K,), jnp.int32),
        pltpu.VMEM((K,), jnp.int32),
        pltpu.VMEM((K, FH), jnp.float32),
        pltpu.VMEM((K, FH), jnp.float32),
        pltpu.VMEM_SHARED((NPAD, FH), jnp.float32),
        pltpu.SemaphoreType.DMA,
        pltpu.SemaphoreType.DMA,
    ],
)
def _edge_scatter(t_hbm, ei_hbm, et_hbm, zeros_hbm, out_hbm,
                  srcv, etv, dstvA, dstvB, idxvA, idxvB, rowsA, rowsB,
                  acc, semA, semB):
    c = lax.axis_index("c")
    s = lax.axis_index("s")

    pltpu.sync_copy(zeros_hbm.at[pl.ds(s * ROWS_PER_TILE, ROWS_PER_TILE)],
                    acc.at[pl.ds(s * ROWS_PER_TILE, ROWS_PER_TILE)])
    plsc.subcore_barrier()

    base = s * EPS

    def load_idx(b, idxv, dstv):
        off = base + b * K
        pltpu.sync_copy(ei_hbm.at[0, pl.ds(off, K)], srcv)
        pltpu.sync_copy(et_hbm.at[pl.ds(off, K)], etv)
        pltpu.sync_copy(ei_hbm.at[1, pl.ds(off, K)], dstv)
        for j in range(K // 16):
            sl = pl.ds(j * 16, 16)
            idxv[sl] = (etv[sl] * N + srcv[sl]) * 2 + c

    load_idx(0, idxvA, dstvA)
    gA = pltpu.async_copy(t_hbm.at[idxvA], rowsA, semA)

    def pair(i, carry):
        b0 = 2 * i
        load_idx(b0 + 1, idxvB, dstvB)
        pltpu.async_copy(t_hbm.at[idxvB], rowsB, semB)
        pltpu.make_async_copy(t_hbm.at[idxvA], rowsA, semA).wait()
        pltpu.sync_copy(rowsA, acc.at[dstvA], add=True)

        @pl.when(b0 + 2 < NBATCH)
        def _():
            load_idx(b0 + 2, idxvA, dstvA)
            pltpu.async_copy(t_hbm.at[idxvA], rowsA, semA)

        pltpu.make_async_copy(t_hbm.at[idxvB], rowsB, semB).wait()
        pltpu.sync_copy(rowsB, acc.at[dstvB], add=True)
        return carry

    lax.fori_loop(0, NBATCH // 2, pair, 0)

    plsc.subcore_barrier()
    pltpu.sync_copy(acc.at[pl.ds(s * ROWS_PER_TILE, ROWS_PER_TILE)],
                    out_hbm.at[c, pl.ds(s * ROWS_PER_TILE, ROWS_PER_TILE)])


def _final_body(p_ref, feat_ref, lw_ref, bias_ref, out_ref):
    h = jnp.dot(feat_ref[...], lw_ref[...], preferred_element_type=jnp.float32)
    agg = jnp.concatenate([p_ref[0], p_ref[1]], axis=1)
    out_ref[...] = h + agg + bias_ref[...]


def _finalize(partials, feat, loop_weight, h_bias):
    return pl.pallas_call(
        _final_body,
        grid=(NT,),
        in_specs=[
            pl.BlockSpec((NC, TN, FH), lambda n: (0, n, 0)),
            pl.BlockSpec((TN, F), lambda n: (n, 0)),
            pl.BlockSpec((F, F), lambda n: (0, 0)),
            pl.BlockSpec((1, F), lambda n: (0, 0)),
        ],
        out_specs=pl.BlockSpec((TN, F), lambda n: (n, 0)),
        out_shape=jax.ShapeDtypeStruct((N, F), jnp.float32),
    )(partials, feat, loop_weight, h_bias.reshape(1, F))


def kernel(feat, edge_index, etypes, W, coeff, h_bias, loop_weight):
    zeros = jnp.zeros((NPAD, FH), jnp.float32)
    table = _build_table(feat, W, coeff)
    table64 = table.reshape(2 * R * N, FH)
    partials = _edge_scatter(table64, edge_index, etypes, zeros)
    return _finalize(partials, feat, loop_weight, h_bias)


# speedup vs baseline: 24.6457x; 1.1813x over previous
import functools

import jax
import jax.numpy as jnp
from jax import lax
from jax.experimental import pallas as pl
from jax.experimental.pallas import tpu as pltpu
from jax.experimental.pallas import tpu_sc as plsc

N = 10000
E = 320000
F = 128
FH = F // 2
R = 16
NB = 4

NC = 2
NS = 16
EPS = E // NS
K = 400
NBATCH = EPS // K
NPAD = 10112
ROWS_PER_TILE = NPAD // NS

TN = 2000
NT = N // TN


def _table_body(coeff_ref, feat_ref, w_ref, out_ref):
    r = pl.program_id(1)
    wr = coeff_ref[r, 0] * w_ref[0]
    for b in range(1, NB):
        wr = wr + coeff_ref[r, b] * w_ref[b]
    out_ref[...] = jnp.dot(feat_ref[...], wr, preferred_element_type=jnp.float32)


def _build_table(feat, W, coeff):
    return pl.pallas_call(
        _table_body,
        grid=(NT, R),
        in_specs=[
            pl.BlockSpec(memory_space=pltpu.SMEM),
            pl.BlockSpec((TN, F), lambda n, r: (n, 0)),
            pl.BlockSpec((NB, F, F), lambda n, r: (0, 0, 0)),
        ],
        out_specs=pl.BlockSpec((TN, F), lambda n, r: (r * NT + n, 0)),
        out_shape=jax.ShapeDtypeStruct((R * N, F), jnp.float32),
    )(coeff, feat, W)


_MESH = plsc.VectorSubcoreMesh(core_axis_name="c", subcore_axis_name="s")


@functools.partial(
    pl.kernel,
    mesh=_MESH,
    compiler_params=pltpu.CompilerParams(use_tc_tiling_on_sc=False),
    out_type=jax.ShapeDtypeStruct((NPAD, F), jnp.float32),
    scratch_types=[
        pltpu.VMEM((K,), jnp.int32),
        pltpu.VMEM((K,), jnp.int32),
        [pltpu.VMEM((K,), jnp.int32)] * 3,
        [pltpu.VMEM((K,), jnp.int32)] * 3,
        [pltpu.VMEM((K, FH), jnp.float32)] * 3,
        pltpu.VMEM_SHARED((NPAD, FH), jnp.float32),
        [pltpu.SemaphoreType.DMA] * 3,
        [pltpu.SemaphoreType.DMA] * 3,
    ],
)
def _edge_scatter(t_hbm, ei_hbm, et_hbm, zeros_hbm, out_hbm,
                  srcv, etv, dstv, idxv, rows, acc, semG, semS):
    c = lax.axis_index("c")
    s = lax.axis_index("s")

    pltpu.sync_copy(zeros_hbm.at[pl.ds(s * ROWS_PER_TILE, ROWS_PER_TILE)],
                    acc.at[pl.ds(s * ROWS_PER_TILE, ROWS_PER_TILE)])
    plsc.subcore_barrier()

    base = s * EPS

    def load_and_gather(b, k):
        off = base + b * K
        pltpu.sync_copy(ei_hbm.at[0, pl.ds(off, K)], srcv)
        pltpu.sync_copy(et_hbm.at[pl.ds(off, K)], etv)
        pltpu.sync_copy(ei_hbm.at[1, pl.ds(off, K)], dstv[k])
        for j in range(K // 16):
            sl = pl.ds(j * 16, 16)
            idxv[k][sl] = (etv[sl] * N + srcv[sl]) * 2 + c
        pltpu.async_copy(t_hbm.at[idxv[k]], rows[k], semG[k])

    def wait_gather(k):
        pltpu.make_async_copy(t_hbm.at[idxv[k]], rows[k], semG[k]).wait()

    def drain_scatter(k):
        pltpu.make_async_copy(rows[k], acc.at[dstv[k]], semS[k]).wait()

    load_and_gather(0, 0)
    load_and_gather(1, 1)

    def step(i, carry):
        for t in range(3):
            b = 3 * i + t
            k = t
            kn = (t + 2) % 3
            wait_gather(k)
            pltpu.async_copy(rows[k], acc.at[dstv[k]], semS[k], add=True)

            @pl.when(b >= 1)
            def _():
                drain_scatter(kn)

            load_and_gather(b + 2, kn)
        return carry

    lax.fori_loop(0, NBATCH // 3, step, 0)

    for b in (NBATCH - 2, NBATCH - 1):
        k = b % 3
        wait_gather(k)
        pltpu.async_copy(rows[k], acc.at[dstv[k]], semS[k], add=True)
    for k in range(3):
        drain_scatter(k)

    plsc.subcore_barrier()
    pltpu.sync_copy(acc.at[pl.ds(s * ROWS_PER_TILE, ROWS_PER_TILE)],
                    out_hbm.at[pl.ds(s * ROWS_PER_TILE, ROWS_PER_TILE),
                               pl.ds(c * FH, FH)])


def _final_body(p_ref, feat_ref, lw_ref, bias_ref, out_ref):
    h = jnp.dot(feat_ref[...], lw_ref[...], preferred_element_type=jnp.float32)
    out_ref[...] = h + p_ref[...] + bias_ref[...]


def _finalize(partials, feat, loop_weight, h_bias):
    return pl.pallas_call(
        _final_body,
        grid=(NT,),
        in_specs=[
            pl.BlockSpec((TN, F), lambda n: (n, 0)),
            pl.BlockSpec((TN, F), lambda n: (n, 0)),
            pl.BlockSpec((F, F), lambda n: (0, 0)),
            pl.BlockSpec((1, F), lambda n: (0, 0)),
        ],
        out_specs=pl.BlockSpec((TN, F), lambda n: (n, 0)),
        out_shape=jax.ShapeDtypeStruct((N, F), jnp.float32),
    )(partials, feat, loop_weight, h_bias.reshape(1, F))


def kernel(feat, edge_index, etypes, W, coeff, h_bias, loop_weight):
    zeros = jnp.zeros((NPAD, FH), jnp.float32)
    table = _build_table(feat, W, coeff)
    table64 = table.reshape(2 * R * N, FH)
    partials = _edge_scatter(table64, edge_index, etypes, zeros)
    return _finalize(partials, feat, loop_weight, h_bias)


# speedup vs baseline: 25.6593x; 1.0411x over previous
import functools

import jax
import jax.numpy as jnp
from jax import lax
from jax.experimental import pallas as pl
from jax.experimental.pallas import tpu as pltpu
from jax.experimental.pallas import tpu_sc as plsc

N = 10000
E = 320000
F = 128
FH = F // 2
R = 16
NB = 4

NC = 2
NS = 16
EPS = E // NS
K = 400
NBATCH = EPS // K
CH = 3
NCHUNK = (NBATCH - 2) // CH
EPAD = 2 * CH * K
NPAD = 10112
ROWS_PER_TILE = NPAD // NS

TN = 2000
NT = N // TN


def _table_body(coeff_ref, feat_ref, w_ref, out_ref):
    r = pl.program_id(1)
    wr = coeff_ref[r, 0] * w_ref[0]
    for b in range(1, NB):
        wr = wr + coeff_ref[r, b] * w_ref[b]
    out_ref[...] = jnp.dot(feat_ref[...], wr, preferred_element_type=jnp.float32)


def _build_table(feat, W, coeff):
    return pl.pallas_call(
        _table_body,
        grid=(NT, R),
        in_specs=[
            pl.BlockSpec(memory_space=pltpu.SMEM),
            pl.BlockSpec((TN, F), lambda n, r: (n, 0)),
            pl.BlockSpec((NB, F, F), lambda n, r: (0, 0, 0)),
        ],
        out_specs=pl.BlockSpec((TN, F), lambda n, r: (r * NT + n, 0)),
        out_shape=jax.ShapeDtypeStruct((R * N, F), jnp.float32),
    )(coeff, feat, W)


_MESH = plsc.VectorSubcoreMesh(core_axis_name="c", subcore_axis_name="s")


@functools.partial(
    pl.kernel,
    mesh=_MESH,
    compiler_params=pltpu.CompilerParams(use_tc_tiling_on_sc=False),
    out_type=jax.ShapeDtypeStruct((NPAD, F), jnp.float32),
    scratch_types=[
        pltpu.VMEM((2 * CH * K,), jnp.int32),
        pltpu.VMEM((2 * CH * K,), jnp.int32),
        [pltpu.VMEM((K,), jnp.int32)] * 3,
        [pltpu.VMEM((K,), jnp.int32)] * 3,
        [pltpu.VMEM((K, FH), jnp.float32)] * 3,
        pltpu.VMEM_SHARED((NPAD, FH), jnp.float32),
        [pltpu.SemaphoreType.DMA] * 3,
        [pltpu.SemaphoreType.DMA] * 3,
    ],
)
def _edge_scatter(t_hbm, ei_hbm, et_hbm, zeros_hbm, out_hbm,
                  srcv, etv, dstv, idxv, rows, acc, semG, semS):
    c = lax.axis_index("c")
    s = lax.axis_index("s")

    pltpu.sync_copy(zeros_hbm.at[pl.ds(s * ROWS_PER_TILE, ROWS_PER_TILE)],
                    acc.at[pl.ds(s * ROWS_PER_TILE, ROWS_PER_TILE)])
    plsc.subcore_barrier()

    base = s * EPS

    def stage_chunk(g):
        reg = lax.rem(g, 2) * (CH * K)
        off = base + g * (CH * K)
        pltpu.sync_copy(ei_hbm.at[0, pl.ds(off, CH * K)],
                        srcv.at[pl.ds(reg, CH * K)])
        pltpu.sync_copy(et_hbm.at[pl.ds(off, CH * K)],
                        etv.at[pl.ds(reg, CH * K)])

    def load_and_gather(b, k, reg):
        pltpu.sync_copy(ei_hbm.at[1, pl.ds(base + b * K, K)], dstv[k])
        soff = reg + k * K
        for j in range(K // 16):
            sl = pl.ds(soff + j * 16, 16)
            idxv[k][pl.ds(j * 16, 16)] = (etv[sl] * N + srcv[sl]) * 2 + c
        pltpu.async_copy(t_hbm.at[idxv[k]], rows[k], semG[k])

    def wait_gather(k):
        pltpu.make_async_copy(t_hbm.at[idxv[k]], rows[k], semG[k]).wait()

    def drain_scatter(k):
        pltpu.make_async_copy(rows[k], acc.at[dstv[k]], semS[k]).wait()

    stage_chunk(0)
    load_and_gather(0, 0, 0)
    load_and_gather(1, 1, 0)

    def step(g, carry):
        stage_chunk(g + 1)
        for t in range(CH):
            b = CH * g + t
            k = t
            kn = (t + 2) % CH
            reg2 = lax.rem(g + (0 if t == 0 else 1), 2) * (CH * K)
            wait_gather(k)
            pltpu.async_copy(rows[k], acc.at[dstv[k]], semS[k], add=True)

            @pl.when(b >= 1)
            def _():
                drain_scatter(kn)

            load_and_gather(b + 2, kn, reg2)
        return carry

    lax.fori_loop(0, NCHUNK, step, 0)

    for b in (NBATCH - 2, NBATCH - 1):
        k = b % CH
        wait_gather(k)
        pltpu.async_copy(rows[k], acc.at[dstv[k]], semS[k], add=True)
    for k in range(CH):
        drain_scatter(k)

    plsc.subcore_barrier()
    pltpu.sync_copy(acc.at[pl.ds(s * ROWS_PER_TILE, ROWS_PER_TILE)],
                    out_hbm.at[pl.ds(s * ROWS_PER_TILE, ROWS_PER_TILE),
                               pl.ds(c * FH, FH)])


def _final_body(p_ref, feat_ref, lw_ref, bias_ref, out_ref):
    h = jnp.dot(feat_ref[...], lw_ref[...], preferred_element_type=jnp.float32)
    out_ref[...] = h + p_ref[...] + bias_ref[...]


def _finalize(partials, feat, loop_weight, h_bias):
    return pl.pallas_call(
        _final_body,
        grid=(NT,),
        in_specs=[
            pl.BlockSpec((TN, F), lambda n: (n, 0)),
            pl.BlockSpec((TN, F), lambda n: (n, 0)),
            pl.BlockSpec((F, F), lambda n: (0, 0)),
            pl.BlockSpec((1, F), lambda n: (0, 0)),
        ],
        out_specs=pl.BlockSpec((TN, F), lambda n: (n, 0)),
        out_shape=jax.ShapeDtypeStruct((N, F), jnp.float32),
    )(partials, feat, loop_weight, h_bias.reshape(1, F))


def kernel(feat, edge_index, etypes, W, coeff, h_bias, loop_weight):
    zeros = jnp.zeros((NPAD, FH), jnp.float32)
    ei_p = jnp.pad(edge_index, ((0, 0), (0, EPAD)))
    et_p = jnp.pad(etypes, (0, EPAD))
    table = _build_table(feat, W, coeff)
    table64 = table.reshape(2 * R * N, FH)
    partials = _edge_scatter(table64, ei_p, et_p, zeros)
    return _finalize(partials, feat, loop_weight, h_bias)
